# baseline (speedup 1.0000x reference)
import functools

import jax
import jax.numpy as jnp
from jax import lax
from jax.experimental import pallas as pl
from jax.experimental.pallas import tpu as pltpu
from jax.experimental.pallas import tpu_sc as plsc

N = 10000
F_IN = 128
H = 8
CH = 8
B = 16
D = H * CH

NC = 2
NS = 16
NW = NC * NS

N_PAD = 10240
ROWS_PER_TILE = N_PAD // NS

E_FULL = 320000 + N
CB = 256
NCHUNK = 41
T_E = CB * NCHUNK
E_PAD = T_E * NW
JR = CB // 128


def _edge_kernel():
  mesh = plsc.VectorSubcoreMesh(
      core_axis_name="c", subcore_axis_name="s", num_cores=NC, num_subcores=NS
  )

  @functools.partial(
      pl.kernel,
      out_type=(
          jax.ShapeDtypeStruct((NC, N_PAD, 16), jnp.float32),
          jax.ShapeDtypeStruct((NC, N_PAD, D), jnp.float32),
      ),
      mesh=mesh,
      compiler_params=pltpu.CompilerParams(
          use_tc_tiling_on_sc=False, needs_layout_passes=False),
      scratch_types=[
          pltpu.VMEM((T_E,), jnp.int32),
          pltpu.VMEM((2, JR, 128), jnp.int32),
          pltpu.VMEM((2, JR, 128), jnp.int32),
          pltpu.VMEM((2, CB, 16), jnp.float32),
          pltpu.VMEM((2, CB, 16), jnp.float32),
          pltpu.VMEM((2, CB, 16), jnp.float32),
          pltpu.VMEM((2, CB, D), jnp.float32),
          pltpu.VMEM_SHARED((N_PAD, 16), jnp.float32),
          pltpu.VMEM_SHARED((N_PAD, D), jnp.float32),
          pltpu.SemaphoreType.DMA,
          pltpu.SemaphoreType.DMA,
          pltpu.SemaphoreType.DMA,
          pltpu.SemaphoreType.DMA,
      ],
  )
  def edge(tbls, tbld, xw, epk, den_out, acc_out,
           pkv, srcv, dstv, S, Dg, EX, X, den_sp, acc_sp,
           gs0, gs1, ss0, ss1, sem_unused_guard=None):
    del sem_unused_guard
    c = lax.axis_index("c")
    s = lax.axis_index("s")
    wid = s * NC + c
    gsem = (gs0, gs1)
    ssem = (ss0, ss1)

    ebase0 = wid * T_E
    idx_cp = pltpu.async_copy(epk.at[pl.ds(ebase0, T_E)], pkv, gs0)

    QR = ROWS_PER_TILE // 4

    def _z(i, carry):
      for k in range(D // 16):
        X[0, i, pl.ds(k * 16, 16)] = jnp.zeros((16,), jnp.float32)
      EX[0, i, :] = jnp.zeros((16,), jnp.float32)
      return carry

    lax.fori_loop(0, QR, _z, 0, unroll=2)

    rb = s * ROWS_PER_TILE
    for q in range(4):
      pltpu.sync_copy(X.at[0, pl.ds(0, QR)],
                      acc_sp.at[pl.ds(rb + q * QR, QR)])
      pltpu.sync_copy(EX.at[0, pl.ds(0, QR)],
                      den_sp.at[pl.ds(rb + q * QR, QR)])
    idx_cp.wait()
    plsc.subcore_barrier()

    def _unpack(k, p):
      def body(i, carry):
        w = pkv[pl.ds(k * CB + i * 16, 16)]
        srcv[p, i // 8, pl.ds((i % 8) * 16, 16)] = w & jnp.int32(16383)
        dstv[p, i // 8, pl.ds((i % 8) * 16, 16)] = lax.shift_right_logical(
            w, jnp.int32(14))
        return carry
      for i in range(CB // 16):
        body(i, 0)

    def _fire_gathers(k, p):
      cps = []
      for j in range(JR):
        cps.append(pltpu.async_copy(
            tbls.at[srcv.at[p, j]], S.at[p, pl.ds(j * 128, 128)], gsem[p]))
        cps.append(pltpu.async_copy(
            tbld.at[dstv.at[p, j]], Dg.at[p, pl.ds(j * 128, 128)], gsem[p]))
        cps.append(pltpu.async_copy(
            xw.at[srcv.at[p, j]], X.at[p, pl.ds(j * 128, 128)], gsem[p]))
      return cps

    def _compute(p):
      def _ex(e, carry):
        lane = lax.broadcasted_iota(jnp.int32, (16,), 0)
        hi = (lane >= 8).astype(jnp.int32)
        a = S[p, e, :] + Dg[p, e, :]
        a = jnp.maximum(a, a * jnp.float32(0.2))
        v = jnp.exp(a)
        EX[p, e, :] = v
        for j in range(4):
          idx = (2 * j) + hi
          g = jnp.take_along_axis(v, idx, axis=0, mode="promise_in_bounds")
          X[p, e, pl.ds(j * 16, 16)] = X[p, e, pl.ds(j * 16, 16)] * g
        return carry

      lax.fori_loop(0, CB, _ex, 0, unroll=2)

    def _fire_scatters(p):
      cps = []
      for j in range(JR):
        cps.append(pltpu.async_copy(
            X.at[p, pl.ds(j * 128, 128)], acc_sp.at[dstv.at[p, j]],
            ssem[p], add=True))
        cps.append(pltpu.async_copy(
            EX.at[p, pl.ds(j * 128, 128)], den_sp.at[dstv.at[p, j]],
            ssem[p], add=True))
      return cps

    gcps = [None, None]
    scps = [None, None]
    _unpack(0, 0)
    gcps[0] = _fire_gathers(0, 0)
    for k in range(NCHUNK):
      p = k % 2
      q = (k + 1) % 2
      if k + 1 < NCHUNK:
        if scps[q] is not None:
          for cp in scps[q]:
            cp.wait()
          scps[q] = None
        _unpack(k + 1, q)
        gcps[q] = _fire_gathers(k + 1, q)
      for cp in gcps[p]:
        cp.wait()
      _compute(p)
      scps[p] = _fire_scatters(p)

    for pp in range(2):
      if scps[pp] is not None:
        for cp in scps[pp]:
          cp.wait()

    plsc.subcore_barrier()
    for q in range(4):
      r = rb + q * QR
      pltpu.sync_copy(acc_sp.at[pl.ds(r, QR)], X.at[0, pl.ds(0, QR)])
      pltpu.sync_copy(X.at[0, pl.ds(0, QR)], acc_out.at[c, pl.ds(r, QR)])
      pltpu.sync_copy(den_sp.at[pl.ds(r, QR)], EX.at[0, pl.ds(0, QR)])
      pltpu.sync_copy(EX.at[0, pl.ds(0, QR)], den_out.at[c, pl.ds(r, QR)])

  return edge


def _prep_call(x_pad, w, a_s, a_d):
  def body(x_ref, w_ref, as_ref, ad_ref, xw_ref, ts_ref, td_ref):
    xw = jnp.dot(x_ref[...], w_ref[...], preferred_element_type=jnp.float32)
    xw_ref[...] = xw
    ts_ref[...] = jnp.dot(xw, as_ref[...], preferred_element_type=jnp.float32)
    td_ref[...] = jnp.dot(xw, ad_ref[...], preferred_element_type=jnp.float32)

  return pl.pallas_call(
      body,
      out_shape=(
          jax.ShapeDtypeStruct((N_PAD, D), jnp.float32),
          jax.ShapeDtypeStruct((N_PAD, 16), jnp.float32),
          jax.ShapeDtypeStruct((N_PAD, 16), jnp.float32),
      ),
  )(x_pad, w, a_s, a_d)


def _post_call(den_parts, acc_parts, erep, bias, g, b, m, v, w2, a_s2, a_d2):
  def body(den_ref, acc_ref, erep_ref, bias_ref, g_ref, b_ref, m_ref, v_ref,
           w2_ref, as_ref, ad_ref, xw_ref, ts_ref, td_ref):
    den = den_ref[0, :, :H] + den_ref[1, :, :H]
    rden = 1.0 / (den + jnp.float32(1e-16))
    r64 = jnp.dot(rden, erep_ref[...], preferred_element_type=jnp.float32)
    o = (acc_ref[0] + acc_ref[1]) * r64 + bias_ref[...]
    o = g_ref[...] * (o - m_ref[...]) * jax.lax.rsqrt(v_ref[...] + 1e-5) \
        + b_ref[...]
    h = jnp.where(o > 0, o, jnp.exp(o) - 1.0)
    xw = jnp.dot(h, w2_ref[...], preferred_element_type=jnp.float32)
    xw_ref[...] = xw
    ts_ref[...] = jnp.dot(xw, as_ref[...], preferred_element_type=jnp.float32)
    td_ref[...] = jnp.dot(xw, ad_ref[...], preferred_element_type=jnp.float32)

  return pl.pallas_call(
      body,
      out_shape=(
          jax.ShapeDtypeStruct((N_PAD, D), jnp.float32),
          jax.ShapeDtypeStruct((N_PAD, 16), jnp.float32),
          jax.ShapeDtypeStruct((N_PAD, 16), jnp.float32),
      ),
  )(den_parts, acc_parts, erep, bias, g, b, m, v, w2, a_s2, a_d2)


def _final_call(den_parts, acc_parts, erep, bias, g, b, m, v, batch_col,
                fc1_W, fc1_b, f1g, f1b, f1m, f1v,
                fc2_W, fc2_b, f2g, f2b, f2m, f2v, fc3_W, fc3_b):
  def body(den_ref, acc_ref, erep_ref, bias_ref, g_ref, b_ref, m_ref, v_ref,
           batch_ref, w1_ref, b1_ref, g1, b1, m1, v1,
           w2_ref, b2_ref, g2, b2, m2, v2, w3_ref, b3_ref, out_ref,
           pmax_ref):
    den = den_ref[0, :, :H] + den_ref[1, :, :H]
    rden = 1.0 / (den + jnp.float32(1e-16))
    r64 = jnp.dot(rden, erep_ref[...], preferred_element_type=jnp.float32)
    o = (acc_ref[0] + acc_ref[1]) * r64 + bias_ref[...]
    o = g_ref[...] * (o - m_ref[...]) * jax.lax.rsqrt(v_ref[...] + 1e-5) \
        + b_ref[...]
    h = jnp.where(o > 0, o, jnp.exp(o) - 1.0)

    bcol = batch_ref[...]
    gid = lax.broadcasted_iota(jnp.int32, (1, B), 1)
    onehot = (bcol == gid).astype(jnp.float32)
    sm = jnp.einsum('nb,nd->bd', onehot, h,
                    preferred_element_type=jnp.float32)
    cnt = jnp.einsum('nb,nc->bc', onehot, jnp.ones((N_PAD, 1), jnp.float32),
                     preferred_element_type=jnp.float32)
    mean = sm / jnp.maximum(cnt, 1.0)

    def _mx(gb, carry):
      mv = jnp.max(jnp.where(bcol == gb, h, -jnp.inf), axis=0, keepdims=True)
      pmax_ref[pl.ds(gb, 1), :] = jnp.where(jnp.isfinite(mv), mv, 0.0)
      return carry

    lax.fori_loop(0, B, _mx, 0)
    gfeat = jnp.concatenate([mean, pmax_ref[...]], axis=1)

    z = jnp.dot(gfeat, w1_ref[...], preferred_element_type=jnp.float32) \
        + b1_ref[...]
    z = g1[...] * (z - m1[...]) * jax.lax.rsqrt(v1[...] + 1e-5) + b1[...]
    z = jnp.maximum(z, 0.0)
    z = jnp.dot(z, w2_ref[...], preferred_element_type=jnp.float32) \
        + b2_ref[...]
    z = g2[...] * (z - m2[...]) * jax.lax.rsqrt(v2[...] + 1e-5) + b2[...]
    z = jnp.maximum(z, 0.0)
    out_ref[...] = jnp.dot(z, w3_ref[...],
                           preferred_element_type=jnp.float32) + b3_ref[...]

  return pl.pallas_call(
      body,
      out_shape=jax.ShapeDtypeStruct((B, 2), jnp.float32),
      scratch_shapes=[pltpu.VMEM((B, D), jnp.float32)],
  )(den_parts, acc_parts, erep, bias, g, b, m, v, batch_col,
    fc1_W, fc1_b, f1g, f1b, f1m, f1v,
    fc2_W, fc2_b, f2g, f2b, f2m, f2v, fc3_W, fc3_b)


def _att_mat(att):
  eye = jnp.eye(H, dtype=jnp.float32)
  return (eye[:, None, :] * att[:, :, None]).reshape(D, H)


@jax.jit
def kernel(x, edge_index, batch, W1, att_src1, att_dst1, bias1,
           bn1_g, bn1_b, bn1_m, bn1_v,
           W2, att_src2, att_dst2, bias2,
           bn2_g, bn2_b, bn2_m, bn2_v,
           fc1_W, fc1_b, bnf1_g, bnf1_b, bnf1_m, bnf1_v,
           fc2_W, fc2_b, bnf2_g, bnf2_b, bnf2_m, bnf2_v,
           fc3_W, fc3_b):
  x_pad = jnp.pad(x, ((0, N_PAD - N), (0, 0)))
  loop = jnp.arange(N, dtype=jnp.int32)
  src = jnp.concatenate([edge_index[0], loop])
  dst = jnp.concatenate([edge_index[1], loop])
  epk = jnp.pad(src | (dst << 14), (0, E_PAD - E_FULL),
                constant_values=N | (N << 14))
  batch_col = jnp.pad(batch, (0, N_PAD - N),
                      constant_values=B).reshape(N_PAD, 1)

  erep = jnp.kron(jnp.eye(H, dtype=jnp.float32),
                  jnp.ones((1, CH), jnp.float32))
  row = lambda p: p.reshape(1, -1)

  edge_fn = _edge_kernel()
  dup = lambda a: jnp.concatenate([a, a], axis=1)
  a_s1 = dup(_att_mat(att_src1))
  a_d1 = dup(_att_mat(att_dst1))
  a_s2 = dup(_att_mat(att_src2))
  a_d2 = dup(_att_mat(att_dst2))

  xw1, ts1, td1 = _prep_call(x_pad, W1, a_s1, a_d1)
  den1, acc1 = edge_fn(ts1, td1, xw1, epk)
  xw2, ts2, td2 = _post_call(
      den1, acc1, erep, row(bias1), row(bn1_g), row(bn1_b), row(bn1_m),
      row(bn1_v), W2, a_s2, a_d2)

  den2, acc2 = edge_fn(ts2, td2, xw2, epk)

  return _final_call(
      den2, acc2, erep, row(bias2), row(bn2_g), row(bn2_b), row(bn2_m),
      row(bn2_v), batch_col,
      fc1_W, row(fc1_b), row(bnf1_g), row(bnf1_b), row(bnf1_m), row(bnf1_v),
      fc2_W, row(fc2_b), row(bnf2_g), row(bnf2_b), row(bnf2_m), row(bnf2_v),
      fc3_W, row(fc3_b))


# speedup vs baseline: 1.0562x; 1.0562x over previous
import functools

import jax
import jax.numpy as jnp
from jax import lax
from jax.experimental import pallas as pl
from jax.experimental.pallas import tpu as pltpu
from jax.experimental.pallas import tpu_sc as plsc

N = 10000
F_IN = 128
H = 8
CH = 8
B = 16
D = H * CH

NC = 2
NS = 16
NW = NC * NS

N_PAD = 10240
ROWS_PER_TILE = N_PAD // NS

E_FULL = 320000 + N
CB = 256
NCHUNK = 41
T_E = CB * NCHUNK
E_PAD = T_E * NW
JR = CB // 128


def _edge_kernel():
  mesh = plsc.VectorSubcoreMesh(
      core_axis_name="c", subcore_axis_name="s", num_cores=NC, num_subcores=NS
  )

  @functools.partial(
      pl.kernel,
      out_type=(
          jax.ShapeDtypeStruct((NC, N_PAD, 16), jnp.float32),
          jax.ShapeDtypeStruct((NC, N_PAD, D), jnp.float32),
      ),
      mesh=mesh,
      compiler_params=pltpu.CompilerParams(
          use_tc_tiling_on_sc=False, needs_layout_passes=False),
      scratch_types=[
          pltpu.VMEM((T_E,), jnp.int32),
          pltpu.VMEM((2, JR, 128), jnp.int32),
          pltpu.VMEM((2, JR, 128), jnp.int32),
          pltpu.VMEM((2, CB, 16), jnp.float32),
          pltpu.VMEM((2, CB, 16), jnp.float32),
          pltpu.VMEM((2, CB, 16), jnp.float32),
          pltpu.VMEM((2, CB, D), jnp.float32),
          pltpu.VMEM_SHARED((N_PAD, 16), jnp.float32),
          pltpu.VMEM_SHARED((N_PAD, D), jnp.float32),
          pltpu.SemaphoreType.DMA,
          pltpu.SemaphoreType.DMA,
          pltpu.SemaphoreType.DMA,
          pltpu.SemaphoreType.DMA,
      ],
  )
  def edge(tbl, xw, epk, den_out, acc_out,
           pkv, srcv, dstv, S, Dg, EX, X, den_sp, acc_sp,
           gs0, gs1, ss0, ss1, sem_unused_guard=None):
    del sem_unused_guard
    c = lax.axis_index("c")
    s = lax.axis_index("s")
    wid = s * NC + c
    gsem = (gs0, gs1)
    ssem = (ss0, ss1)

    ebase0 = wid * T_E
    idx_cp = pltpu.async_copy(epk.at[pl.ds(ebase0, T_E)], pkv, gs0)

    QR = ROWS_PER_TILE // 4

    def _z(i, carry):
      for k in range(D // 16):
        X[0, i, pl.ds(k * 16, 16)] = jnp.zeros((16,), jnp.float32)
      EX[0, i, :] = jnp.zeros((16,), jnp.float32)
      return carry

    lax.fori_loop(0, QR, _z, 0, unroll=2)

    rb = s * ROWS_PER_TILE
    for q in range(4):
      pltpu.sync_copy(X.at[0, pl.ds(0, QR)],
                      acc_sp.at[pl.ds(rb + q * QR, QR)])
      pltpu.sync_copy(EX.at[0, pl.ds(0, QR)],
                      den_sp.at[pl.ds(rb + q * QR, QR)])
    idx_cp.wait()
    plsc.subcore_barrier()

    def _unpack(k, p):
      def body(i, carry):
        w = pkv[pl.ds(k * CB + i * 16, 16)]
        srcv[p, i // 8, pl.ds((i % 8) * 16, 16)] = w & jnp.int32(16383)
        dstv[p, i // 8, pl.ds((i % 8) * 16, 16)] = lax.shift_right_logical(
            w, jnp.int32(14))
        return carry
      for i in range(CB // 16):
        body(i, 0)

    def _fire_gathers(k, p):
      cps = []
      for j in range(JR):
        cps.append(pltpu.async_copy(
            tbl.at[srcv.at[p, j]], S.at[p, pl.ds(j * 128, 128)], gsem[p]))
        cps.append(pltpu.async_copy(
            tbl.at[dstv.at[p, j]], Dg.at[p, pl.ds(j * 128, 128)], gsem[p]))
        cps.append(pltpu.async_copy(
            xw.at[srcv.at[p, j]], X.at[p, pl.ds(j * 128, 128)], gsem[p]))
      return cps

    def _compute(p):
      def _ex(e, carry):
        lane = lax.broadcasted_iota(jnp.int32, (16,), 0)
        sh_idx = jnp.minimum(lane + 8, 15)
        hi = (lane >= 8).astype(jnp.int32)
        dsh = jnp.take_along_axis(Dg[p, e, :], sh_idx, axis=0,
                                  mode="promise_in_bounds")
        a = S[p, e, :] + dsh
        a = jnp.maximum(a, a * jnp.float32(0.2))
        v = jnp.exp(a)
        EX[p, e, :] = v
        for j in range(4):
          idx = (2 * j) + hi
          g = jnp.take_along_axis(v, idx, axis=0, mode="promise_in_bounds")
          X[p, e, pl.ds(j * 16, 16)] = X[p, e, pl.ds(j * 16, 16)] * g
        return carry

      lax.fori_loop(0, CB, _ex, 0, unroll=2)

    def _fire_scatters(p):
      cps = []
      for j in range(JR):
        cps.append(pltpu.async_copy(
            X.at[p, pl.ds(j * 128, 128)], acc_sp.at[dstv.at[p, j]],
            ssem[p], add=True))
        cps.append(pltpu.async_copy(
            EX.at[p, pl.ds(j * 128, 128)], den_sp.at[dstv.at[p, j]],
            ssem[p], add=True))
      return cps

    gcps = [None, None]
    scps = [None, None]
    _unpack(0, 0)
    gcps[0] = _fire_gathers(0, 0)
    for k in range(NCHUNK):
      p = k % 2
      q = (k + 1) % 2
      if k + 1 < NCHUNK:
        if scps[q] is not None:
          for cp in scps[q]:
            cp.wait()
          scps[q] = None
        _unpack(k + 1, q)
        gcps[q] = _fire_gathers(k + 1, q)
      for cp in gcps[p]:
        cp.wait()
      _compute(p)
      scps[p] = _fire_scatters(p)

    for pp in range(2):
      if scps[pp] is not None:
        for cp in scps[pp]:
          cp.wait()

    plsc.subcore_barrier()
    for q in range(4):
      r = rb + q * QR
      pltpu.sync_copy(acc_sp.at[pl.ds(r, QR)], X.at[0, pl.ds(0, QR)])
      pltpu.sync_copy(X.at[0, pl.ds(0, QR)], acc_out.at[c, pl.ds(r, QR)])
      pltpu.sync_copy(den_sp.at[pl.ds(r, QR)], EX.at[0, pl.ds(0, QR)])
      pltpu.sync_copy(EX.at[0, pl.ds(0, QR)], den_out.at[c, pl.ds(r, QR)])

  return edge


def _prep_call(x_pad, w, a_cat):
  def body(x_ref, w_ref, a_ref, xw_ref, t_ref):
    xw = jnp.dot(x_ref[...], w_ref[...], preferred_element_type=jnp.float32)
    xw_ref[...] = xw
    t_ref[...] = jnp.dot(xw, a_ref[...], preferred_element_type=jnp.float32)

  return pl.pallas_call(
      body,
      out_shape=(
          jax.ShapeDtypeStruct((N_PAD, D), jnp.float32),
          jax.ShapeDtypeStruct((N_PAD, 16), jnp.float32),
      ),
  )(x_pad, w, a_cat)


def _post_call(den_parts, acc_parts, erep, bias, g, b, m, v, w2, a_cat2):
  def body(den_ref, acc_ref, erep_ref, bias_ref, g_ref, b_ref, m_ref, v_ref,
           w2_ref, a_ref, xw_ref, t_ref):
    den = den_ref[0, :, :H] + den_ref[1, :, :H]
    rden = 1.0 / (den + jnp.float32(1e-16))
    r64 = jnp.dot(rden, erep_ref[...], preferred_element_type=jnp.float32)
    o = (acc_ref[0] + acc_ref[1]) * r64 + bias_ref[...]
    o = g_ref[...] * (o - m_ref[...]) * jax.lax.rsqrt(v_ref[...] + 1e-5) \
        + b_ref[...]
    h = jnp.where(o > 0, o, jnp.exp(o) - 1.0)
    xw = jnp.dot(h, w2_ref[...], preferred_element_type=jnp.float32)
    xw_ref[...] = xw
    t_ref[...] = jnp.dot(xw, a_ref[...], preferred_element_type=jnp.float32)

  return pl.pallas_call(
      body,
      out_shape=(
          jax.ShapeDtypeStruct((N_PAD, D), jnp.float32),
          jax.ShapeDtypeStruct((N_PAD, 16), jnp.float32),
      ),
  )(den_parts, acc_parts, erep, bias, g, b, m, v, w2, a_cat2)


def _final_call(den_parts, acc_parts, erep, bias, g, b, m, v, batch_col,
                fc1_W, fc1_b, f1g, f1b, f1m, f1v,
                fc2_W, fc2_b, f2g, f2b, f2m, f2v, fc3_W, fc3_b):
  def body(den_ref, acc_ref, erep_ref, bias_ref, g_ref, b_ref, m_ref, v_ref,
           batch_ref, w1_ref, b1_ref, g1, b1, m1, v1,
           w2_ref, b2_ref, g2, b2, m2, v2, w3_ref, b3_ref, out_ref,
           pmax_ref):
    den = den_ref[0, :, :H] + den_ref[1, :, :H]
    rden = 1.0 / (den + jnp.float32(1e-16))
    r64 = jnp.dot(rden, erep_ref[...], preferred_element_type=jnp.float32)
    o = (acc_ref[0] + acc_ref[1]) * r64 + bias_ref[...]
    o = g_ref[...] * (o - m_ref[...]) * jax.lax.rsqrt(v_ref[...] + 1e-5) \
        + b_ref[...]
    h = jnp.where(o > 0, o, jnp.exp(o) - 1.0)

    bcol = batch_ref[...]
    gid = lax.broadcasted_iota(jnp.int32, (1, B), 1)
    onehot = (bcol == gid).astype(jnp.float32)
    sm = jnp.einsum('nb,nd->bd', onehot, h,
                    preferred_element_type=jnp.float32)
    cnt = jnp.einsum('nb,nc->bc', onehot, jnp.ones((N_PAD, 1), jnp.float32),
                     preferred_element_type=jnp.float32)
    mean = sm / jnp.maximum(cnt, 1.0)

    def _mx(gb, carry):
      mv = jnp.max(jnp.where(bcol == gb, h, -jnp.inf), axis=0, keepdims=True)
      pmax_ref[pl.ds(gb, 1), :] = jnp.where(jnp.isfinite(mv), mv, 0.0)
      return carry

    lax.fori_loop(0, B, _mx, 0)
    gfeat = jnp.concatenate([mean, pmax_ref[...]], axis=1)

    z = jnp.dot(gfeat, w1_ref[...], preferred_element_type=jnp.float32) \
        + b1_ref[...]
    z = g1[...] * (z - m1[...]) * jax.lax.rsqrt(v1[...] + 1e-5) + b1[...]
    z = jnp.maximum(z, 0.0)
    z = jnp.dot(z, w2_ref[...], preferred_element_type=jnp.float32) \
        + b2_ref[...]
    z = g2[...] * (z - m2[...]) * jax.lax.rsqrt(v2[...] + 1e-5) + b2[...]
    z = jnp.maximum(z, 0.0)
    out_ref[...] = jnp.dot(z, w3_ref[...],
                           preferred_element_type=jnp.float32) + b3_ref[...]

  return pl.pallas_call(
      body,
      out_shape=jax.ShapeDtypeStruct((B, 2), jnp.float32),
      scratch_shapes=[pltpu.VMEM((B, D), jnp.float32)],
  )(den_parts, acc_parts, erep, bias, g, b, m, v, batch_col,
    fc1_W, fc1_b, f1g, f1b, f1m, f1v,
    fc2_W, fc2_b, f2g, f2b, f2m, f2v, fc3_W, fc3_b)


def _att_mat(att):
  eye = jnp.eye(H, dtype=jnp.float32)
  return (eye[:, None, :] * att[:, :, None]).reshape(D, H)


@jax.jit
def kernel(x, edge_index, batch, W1, att_src1, att_dst1, bias1,
           bn1_g, bn1_b, bn1_m, bn1_v,
           W2, att_src2, att_dst2, bias2,
           bn2_g, bn2_b, bn2_m, bn2_v,
           fc1_W, fc1_b, bnf1_g, bnf1_b, bnf1_m, bnf1_v,
           fc2_W, fc2_b, bnf2_g, bnf2_b, bnf2_m, bnf2_v,
           fc3_W, fc3_b):
  x_pad = jnp.pad(x, ((0, N_PAD - N), (0, 0)))
  loop = jnp.arange(N, dtype=jnp.int32)
  src = jnp.concatenate([edge_index[0], loop])
  dst = jnp.concatenate([edge_index[1], loop])
  epk = jnp.pad(src | (dst << 14), (0, E_PAD - E_FULL),
                constant_values=N | (N << 14))
  batch_col = jnp.pad(batch, (0, N_PAD - N),
                      constant_values=B).reshape(N_PAD, 1)

  erep = jnp.kron(jnp.eye(H, dtype=jnp.float32),
                  jnp.ones((1, CH), jnp.float32))
  row = lambda p: p.reshape(1, -1)

  edge_fn = _edge_kernel()
  a_cat1 = jnp.concatenate([_att_mat(att_src1), _att_mat(att_dst1)], axis=1)
  a_cat2 = jnp.concatenate([_att_mat(att_src2), _att_mat(att_dst2)], axis=1)

  xw1, t1 = _prep_call(x_pad, W1, a_cat1)
  den1, acc1 = edge_fn(t1, xw1, epk)
  xw2, t2 = _post_call(
      den1, acc1, erep, row(bias1), row(bn1_g), row(bn1_b), row(bn1_m),
      row(bn1_v), W2, a_cat2)

  den2, acc2 = edge_fn(t2, xw2, epk)

  return _final_call(
      den2, acc2, erep, row(bias2), row(bn2_g), row(bn2_b), row(bn2_m),
      row(bn2_v), batch_col,
      fc1_W, row(fc1_b), row(bnf1_g), row(bnf1_b), row(bnf1_m), row(bnf1_v),
      fc2_W, row(fc2_b), row(bnf2_g), row(bnf2_b), row(bnf2_m), row(bnf2_v),
      fc3_W, row(fc3_b))
